# Initial kernel scaffold; baseline (speedup 1.0000x reference)
import jax
import jax.numpy as jnp
from jax.experimental import pallas as pl


def kernel(x, table):
    raise NotImplementedError("write your pallas kernel here")


# speedup vs baseline: 1.0259x; 1.0259x over previous
import functools

import jax
import jax.numpy as jnp
from jax import lax
from jax.experimental import pallas as pl
from jax.experimental.pallas import tpu as pltpu
from jax.experimental.pallas import tpu_sc as plsc

B = 16384
SEQ = 51
T = 50
E = 32
NC = 2
NS = 16
NW = NC * NS

IDX_W = 128
BEH_ROWS = B * T // IDX_W
AD_ROWS = B // IDX_W
BEH_ROWS_PER_W = BEH_ROWS // NW
AD_ROWS_PER_W = AD_ROWS // NW
CHUNK = 8
N_CHUNKS = BEH_ROWS_PER_W // CHUNK
CHUNK_IDX = CHUNK * IDX_W


def _sc_gather(table, beh_idx, ad_idx):
    mesh = plsc.VectorSubcoreMesh(
        core_axis_name="c", subcore_axis_name="s", num_cores=NC, num_subcores=NS
    )

    @functools.partial(
        pl.kernel,
        mesh=mesh,
        compiler_params=pltpu.CompilerParams(use_tc_tiling_on_sc=False),
        out_type=(
            jax.ShapeDtypeStruct((B * T, E), jnp.float32),
            jax.ShapeDtypeStruct((B, E), jnp.float32),
        ),
        scratch_types=[
            pltpu.VMEM((CHUNK, IDX_W), jnp.int32),
            pltpu.VMEM((CHUNK_IDX, E), jnp.float32),
            pltpu.VMEM((AD_ROWS_PER_W, IDX_W), jnp.int32),
            pltpu.VMEM((AD_ROWS_PER_W * IDX_W, E), jnp.float32),
            pltpu.SemaphoreType.DMA,
        ],
    )
    def k(table_hbm, beh_hbm, ad_hbm, ub_hbm, qa_hbm,
          idx_v, rows_v, ad_idx_v, ad_rows_v, sem):
        wid = lax.axis_index("s") * NC + lax.axis_index("c")

        pltpu.sync_copy(ad_hbm.at[pl.ds(wid * AD_ROWS_PER_W, AD_ROWS_PER_W)],
                        ad_idx_v)
        for j in range(AD_ROWS_PER_W):
            pltpu.async_copy(table_hbm.at[ad_idx_v.at[j]],
                             ad_rows_v.at[pl.ds(j * IDX_W, IDX_W)], sem).wait()
        pltpu.sync_copy(ad_rows_v,
                        qa_hbm.at[pl.ds(wid * AD_ROWS_PER_W * IDX_W,
                                        AD_ROWS_PER_W * IDX_W)])

        for g in range(N_CHUNKS):
            r0 = wid * BEH_ROWS_PER_W + g * CHUNK
            pltpu.sync_copy(beh_hbm.at[pl.ds(r0, CHUNK)], idx_v)
            for j in range(CHUNK):
                pltpu.async_copy(table_hbm.at[idx_v.at[j]],
                                 rows_v.at[pl.ds(j * IDX_W, IDX_W)], sem).wait()
            out0 = wid * BEH_ROWS_PER_W * IDX_W + g * CHUNK_IDX
            pltpu.sync_copy(rows_v, ub_hbm.at[pl.ds(out0, CHUNK_IDX)])

    return k(table, beh_idx, ad_idx)


def _mask_body(x_ref, o_ref):
    o_ref[...] = (x_ref[...] > 0).astype(jnp.float32)


def _tc_mask(beh_idx):
    blk = 256
    return pl.pallas_call(
        _mask_body,
        grid=(BEH_ROWS // blk,),
        in_specs=[pl.BlockSpec((blk, IDX_W), lambda i: (i, 0))],
        out_specs=pl.BlockSpec((blk, IDX_W), lambda i: (i, 0)),
        out_shape=jax.ShapeDtypeStruct((BEH_ROWS, IDX_W), jnp.float32),
    )(beh_idx)


def kernel(x, table):
    beh = x[:, :T].reshape(BEH_ROWS, IDX_W)
    ads = x[:, T].reshape(AD_ROWS, IDX_W)
    ub, qa = _sc_gather(table, beh, ads)
    mask = _tc_mask(beh)
    return (qa.reshape(B, 1, E),
            ub.reshape(B, T, E),
            mask.reshape(B, T, 1))


# speedup vs baseline: 1.1103x; 1.0823x over previous
import functools

import jax
import jax.numpy as jnp
from jax import lax
from jax.experimental import pallas as pl
from jax.experimental.pallas import tpu as pltpu
from jax.experimental.pallas import tpu_sc as plsc

B = 16384
SEQ = 51
T = 50
E = 32
NC = 2
NS = 16
NW = NC * NS

IDX_W = 128
BEH_ROWS = B * T // IDX_W
AD_ROWS = B // IDX_W
BEH_ROWS_PER_W = BEH_ROWS // NW
AD_ROWS_PER_W = AD_ROWS // NW
CHUNK = 8
N_CHUNKS = BEH_ROWS_PER_W // CHUNK
CHUNK_IDX = CHUNK * IDX_W


def _sc_gather(table, beh_idx, ad_idx):
    mesh = plsc.VectorSubcoreMesh(
        core_axis_name="c", subcore_axis_name="s", num_cores=NC, num_subcores=NS
    )

    @functools.partial(
        pl.kernel,
        mesh=mesh,
        compiler_params=pltpu.CompilerParams(use_tc_tiling_on_sc=False),
        out_type=(
            jax.ShapeDtypeStruct((B * T, E), jnp.float32),
            jax.ShapeDtypeStruct((B, E), jnp.float32),
        ),
        scratch_types=[
            pltpu.VMEM((BEH_ROWS_PER_W, IDX_W), jnp.int32),
            pltpu.VMEM((CHUNK_IDX, E), jnp.float32),
            pltpu.VMEM((CHUNK_IDX, E), jnp.float32),
            pltpu.VMEM((AD_ROWS_PER_W, IDX_W), jnp.int32),
            pltpu.VMEM((AD_ROWS_PER_W * IDX_W, E), jnp.float32),
            pltpu.SemaphoreType.DMA,
            pltpu.SemaphoreType.DMA,
            pltpu.SemaphoreType.DMA,
            pltpu.SemaphoreType.DMA,
            pltpu.SemaphoreType.DMA,
            pltpu.SemaphoreType.DMA,
        ],
    )
    def k(table_hbm, beh_hbm, ad_hbm, ub_hbm, qa_hbm,
          idx_all, rows0, rows1, ad_idx_v, ad_rows_v,
          gsem0, gsem1, osem0, osem1, adsem, aosem):
        wid = lax.axis_index("s") * NC + lax.axis_index("c")
        rows = (rows0, rows1)
        gsem = (gsem0, gsem1)
        osem = (osem0, osem1)

        pltpu.sync_copy(beh_hbm.at[pl.ds(wid * BEH_ROWS_PER_W, BEH_ROWS_PER_W)],
                        idx_all)
        pltpu.sync_copy(ad_hbm.at[pl.ds(wid * AD_ROWS_PER_W, AD_ROWS_PER_W)],
                        ad_idx_v)

        ad_descs = [
            pltpu.async_copy(table_hbm.at[ad_idx_v.at[j]],
                             ad_rows_v.at[pl.ds(j * IDX_W, IDX_W)], adsem)
            for j in range(AD_ROWS_PER_W)
        ]

        def fire(g):
            b = g % 2
            return [
                pltpu.async_copy(table_hbm.at[idx_all.at[g * CHUNK + j]],
                                 rows[b].at[pl.ds(j * IDX_W, IDX_W)], gsem[b])
                for j in range(CHUNK)
            ]

        out_descs = [None, None]
        gather_descs = fire(0)
        for g in range(N_CHUNKS):
            b = g % 2
            next_descs = None
            if g + 1 < N_CHUNKS:
                if out_descs[(g + 1) % 2] is not None:
                    out_descs[(g + 1) % 2].wait()
                    out_descs[(g + 1) % 2] = None
                next_descs = fire(g + 1)
            for d in gather_descs:
                d.wait()
            gather_descs = next_descs
            out0 = wid * BEH_ROWS_PER_W * IDX_W + g * CHUNK_IDX
            out_descs[b] = pltpu.async_copy(
                rows[b], ub_hbm.at[pl.ds(out0, CHUNK_IDX)], osem[b])
        for d in out_descs:
            if d is not None:
                d.wait()

        for d in ad_descs:
            d.wait()
        pltpu.async_copy(ad_rows_v,
                         qa_hbm.at[pl.ds(wid * AD_ROWS_PER_W * IDX_W,
                                         AD_ROWS_PER_W * IDX_W)], aosem).wait()

    return k(table, beh_idx, ad_idx)


def _mask_body(x_ref, o_ref):
    o_ref[...] = (x_ref[...] > 0).astype(jnp.float32)


def _tc_mask(beh_idx):
    blk = 256
    return pl.pallas_call(
        _mask_body,
        grid=(BEH_ROWS // blk,),
        in_specs=[pl.BlockSpec((blk, IDX_W), lambda i: (i, 0))],
        out_specs=pl.BlockSpec((blk, IDX_W), lambda i: (i, 0)),
        out_shape=jax.ShapeDtypeStruct((BEH_ROWS, IDX_W), jnp.float32),
    )(beh_idx)


def kernel(x, table):
    beh = x[:, :T].reshape(BEH_ROWS, IDX_W)
    ads = x[:, T].reshape(AD_ROWS, IDX_W)
    ub, qa = _sc_gather(table, beh, ads)
    mask = _tc_mask(beh)
    return (qa.reshape(B, 1, E),
            ub.reshape(B, T, E),
            mask.reshape(B, T, 1))
